# baseline (speedup 1.0000x reference)
import math

import jax
import jax.numpy as jnp
from jax.experimental import pallas as pl
from jax.experimental.pallas import tpu as pltpu

_Z_COEF = 0.001
_AUX_COEF = 0.01
_CAP_FACTOR = 1.0
_MIN_CAP = 4


def _router_body(x_ref, wt_ref, comb_ref, probs_ref, dw_ref, aux_ref,
                 cnt_ref, psum_ref, z_ref):
    i = pl.program_id(0)
    nblk = pl.num_programs(0)
    TN = x_ref.shape[0]
    E = wt_ref.shape[1]
    C = comb_ref.shape[1]
    N = TN * nblk

    @pl.when(i == 0)
    def _init():
        cnt_ref[...] = jnp.zeros_like(cnt_ref)
        psum_ref[...] = jnp.zeros_like(psum_ref)
        z_ref[0, 0] = 0.0

    logits = jnp.dot(x_ref[...], wt_ref[...],
                     preferred_element_type=jnp.float32)
    lt = logits.T
    m = jnp.max(lt, axis=0, keepdims=True)
    ex = jnp.exp(lt - m)
    s = jnp.sum(ex, axis=0, keepdims=True)
    probs = ex / s
    probs_ref[...] = probs
    lse = m + jnp.log(s)
    z_ref[0, 0] += jnp.sum(lse * lse)

    eio = jax.lax.broadcasted_iota(jnp.int32, (E, TN), 0)
    idx = jnp.min(jnp.where(lt == m, eio, E), axis=0, keepdims=True)
    rw = jnp.max(probs, axis=0, keepdims=True)
    ohe = (eio == idx).astype(jnp.float32)

    r_i = jax.lax.broadcasted_iota(jnp.int32, (TN, TN), 0)
    c_i = jax.lax.broadcasted_iota(jnp.int32, (TN, TN), 1)
    tri = (r_i <= c_i).astype(jnp.float32)
    incl = jnp.dot(ohe, tri, preferred_element_type=jnp.float32)

    cnt = cnt_ref[...]
    pos = jnp.sum((incl + cnt) * ohe, axis=0, keepdims=True) - 1.0
    cnt_ref[...] = cnt + jnp.sum(ohe, axis=1, keepdims=True)
    psum_ref[...] += jnp.sum(probs, axis=1, keepdims=True)

    posi = pos.astype(jnp.int32)
    posk = jnp.where(posi < C, posi, -1)
    eio3 = jax.lax.broadcasted_iota(jnp.int32, (E, C, TN), 0)
    cio3 = jax.lax.broadcasted_iota(jnp.int32, (E, C, TN), 1)
    hit = (eio3 == idx[:, None, :]) & (cio3 == posk[:, None, :])
    comb_ref[...] = jnp.where(hit, rw[:, None, :], 0.0)

    tgt = jnp.where(posk >= 0, idx * C + posk, -1)
    tgtq = tgt >> 2
    shl = jnp.left_shift(jnp.int32(1), 8 * (tgt & 3))
    e4 = jax.lax.broadcasted_iota(jnp.int32, (E, C // 32, 8, TN), 0)
    t4 = jax.lax.broadcasted_iota(jnp.int32, (E, C // 32, 8, TN), 1)
    g4 = jax.lax.broadcasted_iota(jnp.int32, (E, C // 32, 8, TN), 2)
    q4 = e4 * 16 + t4 * 8 + g4
    dw_ref[...] = jnp.where(q4 == tgtq[0][None, None, None, :],
                            shl[0][None, None, None, :], 0)

    @pl.when(i == nblk - 1)
    def _fin():
        fi_pi = jnp.sum(cnt_ref[...] * psum_ref[...]) / (N * N)
        aux_ref[0, 0] = (_AUX_COEF * E * fi_pi
                         + _Z_COEF * (z_ref[0, 0] / N))


def kernel(x, W):
    B, T, D = x.shape
    N = B * T
    E = W.shape[0]
    C = max(int(math.ceil(_CAP_FACTOR * N / E)), _MIN_CAP)
    TN = 512
    nblk = N // TN

    xf = x.reshape(N, D)
    wt = W.T

    comb_t, probs_t, dw, aux = pl.pallas_call(
        _router_body,
        grid=(nblk,),
        in_specs=[
            pl.BlockSpec((TN, D), lambda i: (i, 0)),
            pl.BlockSpec((D, E), lambda i: (0, 0)),
        ],
        out_specs=[
            pl.BlockSpec((E, C, TN), lambda i: (0, 0, i)),
            pl.BlockSpec((E, TN), lambda i: (0, i)),
            pl.BlockSpec((E, C // 32, 8, TN), lambda i: (0, 0, 0, i)),
            pl.BlockSpec(memory_space=pltpu.SMEM),
        ],
        out_shape=[
            jax.ShapeDtypeStruct((E, C, N), jnp.float32),
            jax.ShapeDtypeStruct((E, N), jnp.float32),
            jax.ShapeDtypeStruct((E, C // 32, 8, N), jnp.int32),
            jax.ShapeDtypeStruct((1, 1), jnp.float32),
        ],
        scratch_shapes=[
            pltpu.VMEM((E, 1), jnp.float32),
            pltpu.VMEM((E, 1), jnp.float32),
            pltpu.SMEM((1, 1), jnp.float32),
        ],
        compiler_params=pltpu.CompilerParams(
            dimension_semantics=("arbitrary",),
        ),
    )(xf, wt)

    db = jax.lax.bitcast_convert_type(dw, jnp.int8)
    dispatch_mask = (db.transpose(3, 0, 1, 2, 4)
                     .reshape(N, E, C).astype(jnp.bool_))
    combine_weights = jnp.transpose(comb_t, (2, 0, 1))
    return dispatch_mask, combine_weights, aux[0, 0], probs_t.T


# speedup vs baseline: 3.0572x; 3.0572x over previous
import math

import jax
import jax.numpy as jnp
from jax.experimental import pallas as pl
from jax.experimental.pallas import tpu as pltpu

_Z_COEF = 0.001
_AUX_COEF = 0.01
_CAP_FACTOR = 1.0
_MIN_CAP = 4


def _router_body(x_ref, wt_ref, comb_ref, probs_ref, tgt_ref, aux_ref,
                 cnt_ref, psum_ref, z_ref):
    i = pl.program_id(0)
    nblk = pl.num_programs(0)
    TN = x_ref.shape[0]
    E = wt_ref.shape[1]
    C = comb_ref.shape[1]
    N = TN * nblk

    @pl.when(i == 0)
    def _init():
        cnt_ref[...] = jnp.zeros_like(cnt_ref)
        psum_ref[...] = jnp.zeros_like(psum_ref)
        z_ref[0, 0] = 0.0

    logits = jnp.dot(x_ref[...], wt_ref[...],
                     preferred_element_type=jnp.float32)
    lt = logits.T
    m = jnp.max(lt, axis=0, keepdims=True)
    ex = jnp.exp(lt - m)
    s = jnp.sum(ex, axis=0, keepdims=True)
    probs = ex / s
    probs_ref[...] = probs
    lse = m + jnp.log(s)
    z_ref[0, 0] += jnp.sum(lse * lse)

    eio = jax.lax.broadcasted_iota(jnp.int32, (E, TN), 0)
    idx = jnp.min(jnp.where(lt == m, eio, E), axis=0, keepdims=True)
    rw = jnp.max(probs, axis=0, keepdims=True)
    ohe = (eio == idx).astype(jnp.float32)

    r_i = jax.lax.broadcasted_iota(jnp.int32, (TN, TN), 0)
    c_i = jax.lax.broadcasted_iota(jnp.int32, (TN, TN), 1)
    tri = (r_i <= c_i).astype(jnp.float32)
    incl = jnp.dot(ohe, tri, preferred_element_type=jnp.float32)

    cnt = cnt_ref[...]
    pos = jnp.sum((incl + cnt) * ohe, axis=0, keepdims=True) - 1.0
    cnt_ref[...] = cnt + jnp.sum(ohe, axis=1, keepdims=True)
    psum_ref[...] += jnp.sum(probs, axis=1, keepdims=True)

    posi = pos.astype(jnp.int32)
    posk = jnp.where(posi < C, posi, -1)
    eio3 = jax.lax.broadcasted_iota(jnp.int32, (E, C, TN), 0)
    cio3 = jax.lax.broadcasted_iota(jnp.int32, (E, C, TN), 1)
    hit = (eio3 == idx[:, None, :]) & (cio3 == posk[:, None, :])
    comb_ref[...] = jnp.where(hit, rw[:, None, :], 0.0)
    tgt_ref[...] = jnp.where(posk >= 0, idx * C + posk, -1)

    @pl.when(i == nblk - 1)
    def _fin():
        fi_pi = jnp.sum(cnt_ref[...] * psum_ref[...]) / (N * N)
        aux_ref[0, 0] = (_AUX_COEF * E * fi_pi
                         + _Z_COEF * (z_ref[0, 0] / N))


def kernel(x, W):
    B, T, D = x.shape
    N = B * T
    E = W.shape[0]
    C = max(int(math.ceil(_CAP_FACTOR * N / E)), _MIN_CAP)
    TN = 512
    nblk = N // TN

    xf = x.reshape(N, D)
    wt = W.T

    comb_t, probs_t, tgt, aux = pl.pallas_call(
        _router_body,
        grid=(nblk,),
        in_specs=[
            pl.BlockSpec((TN, D), lambda i: (i, 0)),
            pl.BlockSpec((D, E), lambda i: (0, 0)),
        ],
        out_specs=[
            pl.BlockSpec((E, C, TN), lambda i: (0, 0, i)),
            pl.BlockSpec((E, TN), lambda i: (0, i)),
            pl.BlockSpec((1, TN), lambda i: (0, i)),
            pl.BlockSpec(memory_space=pltpu.SMEM),
        ],
        out_shape=[
            jax.ShapeDtypeStruct((E, C, N), jnp.float32),
            jax.ShapeDtypeStruct((E, N), jnp.float32),
            jax.ShapeDtypeStruct((1, N), jnp.int32),
            jax.ShapeDtypeStruct((1, 1), jnp.float32),
        ],
        scratch_shapes=[
            pltpu.VMEM((E, 1), jnp.float32),
            pltpu.VMEM((E, 1), jnp.float32),
            pltpu.SMEM((1, 1), jnp.float32),
        ],
        compiler_params=pltpu.CompilerParams(
            dimension_semantics=("arbitrary",),
        ),
    )(xf, wt)

    ecgrid = jnp.arange(E * C, dtype=jnp.int32).reshape(E, C)
    dispatch_mask = tgt.reshape(N)[:, None, None] == ecgrid[None]
    combine_weights = jnp.transpose(comb_t, (2, 0, 1))
    return dispatch_mask, combine_weights, aux[0, 0], probs_t.T
